# Initial kernel scaffold; baseline (speedup 1.0000x reference)
import jax
import jax.numpy as jnp
from jax.experimental import pallas as pl


def kernel(x, edge_index, batch, W1, b1, W2, b2, fc_W, fc_b):
    raise NotImplementedError("write your pallas kernel here")


# speedup vs baseline: 16.0909x; 16.0909x over previous
import functools

import jax
import jax.numpy as jnp
from jax import lax
from jax.experimental import pallas as pl
from jax.experimental.pallas import tpu as pltpu
from jax.experimental.pallas import tpu_sc as plsc

N = 10000
NP = 10240
E = 320000
D = 128
NC = 2
NS = 16
NW = NC * NS
CH = 128
NG = E // CH
GBASE = NG // NW
GREM = NG % NW
ROWS_PER_SUB = NP // NS
BR = 1024
GRID = NP // BR

_mesh = plsc.VectorSubcoreMesh(core_axis_name="c", subcore_axis_name="s",
                               num_cores=NC, num_subcores=NS)


def _worker_range(w):
    lo = w * GBASE + jnp.minimum(w, GREM)
    hi = lo + GBASE + jnp.where(w < GREM, 1, 0)
    return lo, hi


@functools.partial(
    pl.kernel,
    out_type=jax.ShapeDtypeStruct((NC, NP), jnp.float32),
    mesh=_mesh,
    scratch_types=[
        pltpu.VMEM((CH,), jnp.int32),
        pltpu.VMEM((CH,), jnp.float32),
        pltpu.VMEM((ROWS_PER_SUB,), jnp.float32),
        pltpu.VMEM_SHARED((NP,), jnp.float32),
    ],
)
def _deg_kernel(dst_hbm, degp_hbm, idx_v, ones_v, zeros_v, deg_sh):
    c = lax.axis_index("c")
    s = lax.axis_index("s")
    w = s * NC + c
    for i in range(CH // 16):
        ones_v[pl.ds(i * 16, 16)] = jnp.ones((16,), jnp.float32)
    for i in range(ROWS_PER_SUB // 16):
        zeros_v[pl.ds(i * 16, 16)] = jnp.zeros((16,), jnp.float32)
    pltpu.sync_copy(zeros_v, deg_sh.at[pl.ds(s * ROWS_PER_SUB, ROWS_PER_SUB)])
    plsc.subcore_barrier()

    g_lo, g_hi = _worker_range(w)

    def step(g, carry):
        pltpu.sync_copy(dst_hbm.at[pl.ds(g * CH, CH)], idx_v)
        pltpu.sync_copy(ones_v, deg_sh.at[idx_v], add=True)
        return carry

    lax.fori_loop(g_lo, g_hi, step, 0)
    plsc.subcore_barrier()
    pltpu.sync_copy(deg_sh.at[pl.ds(s * ROWS_PER_SUB, ROWS_PER_SUB)],
                    degp_hbm.at[c, pl.ds(s * ROWS_PER_SUB, ROWS_PER_SUB)])


@functools.partial(
    pl.kernel,
    out_type=jax.ShapeDtypeStruct((NC, NP, D), jnp.float32),
    mesh=_mesh,
    scratch_types=[
        pltpu.VMEM((CH,), jnp.int32),
        pltpu.VMEM((CH,), jnp.int32),
        pltpu.VMEM((CH, D), jnp.float32),
        pltpu.VMEM_SHARED((NP, D), jnp.float32),
        pltpu.SemaphoreType.DMA,
    ],
)
def _msg_kernel(hs_hbm, src_hbm, dst_hbm, out_hbm,
                isrc_v, idst_v, rows_v, acc_sh, gsem):
    c = lax.axis_index("c")
    s = lax.axis_index("s")
    w = s * NC + c
    pltpu.sync_copy(hs_hbm.at[pl.ds(s * ROWS_PER_SUB, ROWS_PER_SUB)],
                    acc_sh.at[pl.ds(s * ROWS_PER_SUB, ROWS_PER_SUB)])
    plsc.subcore_barrier()

    g_lo, g_hi = _worker_range(w)

    def step(g, carry):
        pltpu.sync_copy(src_hbm.at[pl.ds(g * CH, CH)], isrc_v)
        pltpu.sync_copy(dst_hbm.at[pl.ds(g * CH, CH)], idst_v)
        pltpu.async_copy(hs_hbm.at[isrc_v], rows_v, gsem).wait()
        pltpu.sync_copy(rows_v, acc_sh.at[idst_v], add=True)
        return carry

    lax.fori_loop(g_lo, g_hi, step, 0)
    plsc.subcore_barrier()
    pltpu.sync_copy(acc_sh.at[pl.ds(s * ROWS_PER_SUB, ROWS_PER_SUB)],
                    out_hbm.at[c, pl.ds(s * ROWS_PER_SUB, ROWS_PER_SUB)])


def _b1_body(x_ref, w_ref, d0_ref, d1_ref, hs_ref, dinv_ref):
    deg = d0_ref[...] + d1_ref[...] + 1.0
    dinv = lax.rsqrt(deg)
    dinv_ref[...] = dinv
    h = jnp.dot(x_ref[...], w_ref[...], preferred_element_type=jnp.float32)
    hs_ref[...] = h * dinv


def _b1(xp, W1, d0, d1):
    return pl.pallas_call(
        _b1_body,
        grid=(GRID,),
        in_specs=[
            pl.BlockSpec((BR, D), lambda i: (i, 0)),
            pl.BlockSpec((D, D), lambda i: (0, 0)),
            pl.BlockSpec((BR, 1), lambda i: (i, 0)),
            pl.BlockSpec((BR, 1), lambda i: (i, 0)),
        ],
        out_specs=[
            pl.BlockSpec((BR, D), lambda i: (i, 0)),
            pl.BlockSpec((BR, 1), lambda i: (i, 0)),
        ],
        out_shape=[
            jax.ShapeDtypeStruct((NP, D), jnp.float32),
            jax.ShapeDtypeStruct((NP, 1), jnp.float32),
        ],
    )(xp, W1, d0, d1)


def _b2_body(p0_ref, p1_ref, hs_ref, dinv_ref, b_ref, w_ref, out_ref):
    dinv = dinv_ref[...]
    pre = dinv * (p0_ref[...] + p1_ref[...] - hs_ref[...]) + b_ref[...]
    h = jnp.maximum(pre, 0.0)
    out_ref[...] = jnp.dot(h, w_ref[...],
                           preferred_element_type=jnp.float32) * dinv


def _b2(p0, p1, hs, dinv, b, W2):
    return pl.pallas_call(
        _b2_body,
        grid=(GRID,),
        in_specs=[
            pl.BlockSpec((BR, D), lambda i: (i, 0)),
            pl.BlockSpec((BR, D), lambda i: (i, 0)),
            pl.BlockSpec((BR, D), lambda i: (i, 0)),
            pl.BlockSpec((BR, 1), lambda i: (i, 0)),
            pl.BlockSpec((1, D), lambda i: (0, 0)),
            pl.BlockSpec((D, D), lambda i: (0, 0)),
        ],
        out_specs=pl.BlockSpec((BR, D), lambda i: (i, 0)),
        out_shape=jax.ShapeDtypeStruct((NP, D), jnp.float32),
    )(p0, p1, hs, dinv, b, W2)


def _b3_body(p0_ref, p1_ref, hs_ref, dinv_ref, b_ref, w_ref, fb_ref, out_ref):
    pre = dinv_ref[...] * (p0_ref[...] + p1_ref[...] - hs_ref[...]) + b_ref[...]
    h = jnp.maximum(pre, 0.0)
    out_ref[...] = jnp.dot(h, w_ref[...],
                           preferred_element_type=jnp.float32) + fb_ref[...]


def _b3(p0, p1, hs, dinv, b, fcW, fcb):
    return pl.pallas_call(
        _b3_body,
        grid=(GRID,),
        in_specs=[
            pl.BlockSpec((BR, D), lambda i: (i, 0)),
            pl.BlockSpec((BR, D), lambda i: (i, 0)),
            pl.BlockSpec((BR, D), lambda i: (i, 0)),
            pl.BlockSpec((BR, 1), lambda i: (i, 0)),
            pl.BlockSpec((1, D), lambda i: (0, 0)),
            pl.BlockSpec((D, D), lambda i: (0, 0)),
            pl.BlockSpec((1, D), lambda i: (0, 0)),
        ],
        out_specs=pl.BlockSpec((BR, D), lambda i: (i, 0)),
        out_shape=jax.ShapeDtypeStruct((NP, D), jnp.float32),
    )(p0, p1, hs, dinv, b, fcW, fcb)


def kernel(x, edge_index, batch, W1, b1, W2, b2, fc_W, fc_b):
    del batch
    src = edge_index[0]
    dst = edge_index[1]
    xp = jnp.pad(x.astype(jnp.float32), ((0, NP - N), (0, 0)))

    degp = _deg_kernel(dst)
    d0 = degp[0][:, None]
    d1 = degp[1][:, None]

    hs1, dinv = _b1(xp, W1, d0, d1)
    p = _msg_kernel(hs1, src, dst)
    hs2 = _b2(p[0], p[1], hs1, dinv, b1[None, :], W2)
    q = _msg_kernel(hs2, src, dst)
    out = _b3(q[0], q[1], hs2, dinv, b2[None, :], fc_W, fc_b[None, :])
    return out[:N]
